# Initial kernel scaffold; baseline (speedup 1.0000x reference)
import jax
import jax.numpy as jnp
from jax.experimental import pallas as pl


def kernel(hidden_states, P, CB):
    raise NotImplementedError("write your pallas kernel here")


# speedup vs baseline: 6.6295x; 6.6295x over previous
import jax
import jax.numpy as jnp
from jax import lax
from jax.experimental import pallas as pl
from jax.experimental.pallas import tpu as pltpu

_TK = 2048


def _vq_kernel(x_ref, p_ref, cbt_ref, out_ref):
    rows = x_ref.shape[0]
    k_total = cbt_ref.shape[1]

    h = jnp.dot(x_ref[...], p_ref[...], preferred_element_type=jnp.float32)
    n = jnp.sqrt(jnp.sum(h * h, axis=1, keepdims=True))
    hn = h / jnp.maximum(n, 1e-12)

    best = jnp.full((rows, 1), jnp.inf, dtype=jnp.float32)
    bidx = jnp.zeros((rows, 1), dtype=jnp.int32)
    for t in range(k_total // _TK):
        cbt = cbt_ref[:, t * _TK:(t + 1) * _TK]
        cb2 = jnp.sum(cbt * cbt, axis=0, keepdims=True)
        s = jnp.dot(hn, cbt, preferred_element_type=jnp.float32,
                    precision=lax.Precision.HIGHEST)
        d = cb2 - 2.0 * s
        tmin = jnp.min(d, axis=1, keepdims=True)
        ii = lax.broadcasted_iota(jnp.int32, d.shape, 1) + t * _TK
        tidx = jnp.min(jnp.where(d == tmin, ii, k_total), axis=1, keepdims=True)
        upd = tmin < best
        best = jnp.where(upd, tmin, best)
        bidx = jnp.where(upd, tidx, bidx)
    out_ref[...] = bidx


def kernel(hidden_states, P, CB):
    B, T, D = hidden_states.shape
    NB, K, CD = CB.shape
    x = hidden_states.reshape(B * T, D)
    codes = pl.pallas_call(
        _vq_kernel,
        out_shape=jax.ShapeDtypeStruct((B * T, 1), jnp.int32),
    )(x, P[0], CB[0].T)
    return codes.reshape(B, NB, T)


# speedup vs baseline: 7.2884x; 1.0994x over previous
import jax
import jax.numpy as jnp
from jax import lax
from jax.experimental import pallas as pl
from jax.experimental.pallas import tpu as pltpu

_TK = 2048
_L = 128


def _vq_kernel(x_ref, p_ref, cbt_ref, out_ref):
    rows = x_ref.shape[0]
    k_total = cbt_ref.shape[1]

    h = jnp.dot(x_ref[...], p_ref[...], preferred_element_type=jnp.float32)
    n = jnp.sqrt(jnp.sum(h * h, axis=1, keepdims=True))
    hn = h / jnp.maximum(n, 1e-12)

    mval = jnp.full((rows, _L), jnp.inf, dtype=jnp.float32)
    mcol = jnp.zeros((rows, _L), dtype=jnp.int32)
    for t in range(k_total // _TK):
        cbt = cbt_ref[:, t * _TK:(t + 1) * _TK]
        cb2 = jnp.sum(cbt * cbt, axis=0, keepdims=True)
        cbt2 = cbt * (-2.0)
        s = jnp.dot(hn, cbt2, preferred_element_type=jnp.float32,
                    precision=lax.Precision.HIGHEST)
        d = cb2 + s
        for c in range(_TK // _L):
            dc = d[:, c * _L:(c + 1) * _L]
            upd = dc < mval
            mval = jnp.where(upd, dc, mval)
            mcol = jnp.where(upd, t * (_TK // _L) + c, mcol)
    gmin = jnp.min(mval, axis=1, keepdims=True)
    kfull = mcol * _L + lax.broadcasted_iota(jnp.int32, (rows, _L), 1)
    ksel = jnp.where(mval == gmin, kfull, k_total)
    out_ref[...] = jnp.min(ksel, axis=1, keepdims=True)


def kernel(hidden_states, P, CB):
    B, T, D = hidden_states.shape
    NB, K, CD = CB.shape
    x = hidden_states.reshape(B * T, D)
    codes = pl.pallas_call(
        _vq_kernel,
        out_shape=jax.ShapeDtypeStruct((B * T, 1), jnp.int32),
    )(x, P[0], CB[0].T)
    return codes.reshape(B, NB, T)
